# baseline (speedup 1.0000x reference)
import functools

import jax
import jax.numpy as jnp
from jax import lax
from jax.experimental import pallas as pl
from jax.experimental.pallas import tpu as pltpu
from jax.experimental.pallas import tpu_sc as plsc

EMB = 128
VOCAB = 84
NUM_ROWS = 16384 * 200
NUM_WORKERS = 32
ROWS_PER_WORKER = NUM_ROWS // NUM_WORKERS
NBUF = 4
CHUNK = 200
STEPS = ROWS_PER_WORKER // CHUNK
JB = STEPS // NBUF


def _sc_body(idx_hbm, table_hbm, out_hbm, table_sp, *bufs):
    idx = bufs[0:NBUF]
    rows = bufs[NBUF:2 * NBUF]
    isem = bufs[2 * NBUF:3 * NBUF]
    gsem = bufs[3 * NBUF:4 * NBUF]
    ssem = bufs[4 * NBUF:5 * NBUF]

    sid = lax.axis_index("s")
    wid = sid * 2 + lax.axis_index("c")
    base = wid * ROWS_PER_WORKER

    @pl.when(sid == 0)
    def _stage():
        pltpu.sync_copy(table_hbm, rows[0].at[pl.ds(0, VOCAB)])
        pltpu.sync_copy(rows[0].at[pl.ds(0, VOCAB)], table_sp)

    plsc.subcore_barrier()

    def idx_start(i, s):
        pltpu.async_copy(idx_hbm.at[pl.ds(base + i * CHUNK, CHUNK)],
                         idx[s], isem[s])

    def idx_wait(s):
        pltpu.make_async_copy(idx_hbm.at[pl.ds(0, CHUNK)], idx[s], isem[s]).wait()

    def gather_start(s):
        pltpu.async_copy(table_sp.at[idx[s]], rows[s], gsem[s])

    def gather_wait(s):
        pltpu.make_async_copy(table_sp.at[idx[s]], rows[s], gsem[s]).wait()

    def store_start(i, s):
        pltpu.async_copy(rows[s], out_hbm.at[pl.ds(base + i * CHUNK, CHUNK)],
                         ssem[s])

    def store_wait(s):
        pltpu.make_async_copy(rows[s], out_hbm.at[pl.ds(0, CHUNK)], ssem[s]).wait()

    for s in range(NBUF):
        idx_start(s, s)
    idx_wait(0)
    gather_start(0)

    def block(jb, carry):
        i0 = NBUF * jb
        for s in range(NBUF):
            gather_wait(s)
            store_start(i0 + s, s)

            @pl.when(jb < JB - 1)
            def _():
                idx_start(i0 + s + NBUF, s)

            s1 = (s + 1) % NBUF
            if s < NBUF - 1:
                idx_wait(s1)

                @pl.when(jb >= 1)
                def _():
                    store_wait(s1)

                gather_start(s1)
            else:
                @pl.when(jb < JB - 1)
                def _():
                    idx_wait(s1)
                    store_wait(s1)
                    gather_start(s1)

        return carry

    lax.fori_loop(0, JB, block, 0)

    for s in range(NBUF):
        store_wait(s)


_sc_gather = functools.partial(
    pl.kernel,
    mesh=plsc.VectorSubcoreMesh(core_axis_name="c", subcore_axis_name="s"),
    out_type=jax.ShapeDtypeStruct((NUM_ROWS, EMB), jnp.float32),
    scratch_types=(
        [pltpu.VMEM_SHARED((VOCAB, EMB), jnp.float32)]
        + [pltpu.VMEM((CHUNK,), jnp.int32) for _ in range(NBUF)]
        + [pltpu.VMEM((CHUNK, EMB), jnp.float32) for _ in range(NBUF)]
        + [pltpu.SemaphoreType.DMA for _ in range(3 * NBUF)]
    ),
)(_sc_body)


def kernel(x, table):
    flat = _sc_gather(x.reshape(-1), table)
    return flat.reshape(x.shape + (EMB,))


# speedup vs baseline: 1.0030x; 1.0030x over previous
import functools

import jax
import jax.numpy as jnp
from jax import lax
from jax.experimental import pallas as pl
from jax.experimental.pallas import tpu as pltpu
from jax.experimental.pallas import tpu_sc as plsc

EMB = 128
VOCAB = 84
NUM_ROWS = 16384 * 200
NUM_WORKERS = 32
ROWS_PER_WORKER = NUM_ROWS // NUM_WORKERS
NBUF = 2
CHUNK = 400
STEPS = ROWS_PER_WORKER // CHUNK
JB = STEPS // NBUF


def _sc_body(idx_hbm, table_hbm, out_hbm, table_sp, *bufs):
    idx = bufs[0:NBUF]
    rows = bufs[NBUF:2 * NBUF]
    isem = bufs[2 * NBUF:3 * NBUF]
    gsem = bufs[3 * NBUF:4 * NBUF]
    ssem = bufs[4 * NBUF:5 * NBUF]

    sid = lax.axis_index("s")
    wid = sid * 2 + lax.axis_index("c")
    base = wid * ROWS_PER_WORKER

    pltpu.sync_copy(table_hbm, rows[0].at[pl.ds(0, VOCAB)])
    pltpu.sync_copy(rows[0].at[pl.ds(0, VOCAB)],
                    table_sp.at[pl.ds(sid * VOCAB, VOCAB)])
    plsc.subcore_barrier()

    roff = jnp.full((16,), sid * VOCAB, dtype=jnp.int32)

    def idx_adjust(s):
        for k in range(CHUNK // 16):
            sl = pl.ds(k * 16, 16)
            idx[s][sl] = idx[s][sl] + roff

    def idx_start(i, s):
        pltpu.async_copy(idx_hbm.at[pl.ds(base + i * CHUNK, CHUNK)],
                         idx[s], isem[s])

    def idx_wait(s):
        pltpu.make_async_copy(idx_hbm.at[pl.ds(0, CHUNK)], idx[s], isem[s]).wait()

    def gather_start(s):
        pltpu.async_copy(table_sp.at[idx[s]], rows[s], gsem[s])

    def gather_wait(s):
        pltpu.make_async_copy(table_sp.at[idx[s]], rows[s], gsem[s]).wait()

    def store_start(i, s):
        pltpu.async_copy(rows[s], out_hbm.at[pl.ds(base + i * CHUNK, CHUNK)],
                         ssem[s])

    def store_wait(s):
        pltpu.make_async_copy(rows[s], out_hbm.at[pl.ds(0, CHUNK)], ssem[s]).wait()

    for s in range(NBUF):
        idx_start(s, s)
    idx_wait(0)
    idx_adjust(0)
    gather_start(0)

    def block(jb, carry):
        i0 = NBUF * jb
        for s in range(NBUF):
            gather_wait(s)
            store_start(i0 + s, s)

            @pl.when(jb < JB - 1)
            def _():
                idx_start(i0 + s + NBUF, s)

            s1 = (s + 1) % NBUF
            if s < NBUF - 1:
                idx_wait(s1)
                idx_adjust(s1)

                @pl.when(jb >= 1)
                def _():
                    store_wait(s1)

                gather_start(s1)
            else:
                @pl.when(jb < JB - 1)
                def _():
                    idx_wait(s1)
                    idx_adjust(s1)
                    store_wait(s1)
                    gather_start(s1)

        return carry

    lax.fori_loop(0, JB, block, 0)

    for s in range(NBUF):
        store_wait(s)


_sc_gather = functools.partial(
    pl.kernel,
    mesh=plsc.VectorSubcoreMesh(core_axis_name="c", subcore_axis_name="s"),
    out_type=jax.ShapeDtypeStruct((NUM_ROWS, EMB), jnp.float32),
    scratch_types=(
        [pltpu.VMEM_SHARED((16 * VOCAB, EMB), jnp.float32)]
        + [pltpu.VMEM((CHUNK,), jnp.int32) for _ in range(NBUF)]
        + [pltpu.VMEM((CHUNK, EMB), jnp.float32) for _ in range(NBUF)]
        + [pltpu.SemaphoreType.DMA for _ in range(3 * NBUF)]
    ),
)(_sc_body)


def kernel(x, table):
    flat = _sc_gather(x.reshape(-1), table)
    return flat.reshape(x.shape + (EMB,))
